# baseline (speedup 1.0000x reference)
import functools

import jax
import jax.numpy as jnp
from jax import lax
from jax.experimental import pallas as pl
from jax.experimental.pallas import tpu as pltpu
from jax.experimental.pallas import tpu_sc as plsc

EMBEDDING_DIM = 64
PADDED_DIM = 128
NUM_CORES = 2
NUM_SUBCORES = 16
NUM_WORKERS = NUM_CORES * NUM_SUBCORES
NBUF = 2
CHUNK = 400


def kernel(x, weight):
    batch, seq = x.shape
    n = batch * seq
    idx = x.reshape(n)
    w128 = jnp.pad(weight, ((0, 0), (0, PADDED_DIM - EMBEDDING_DIM)))
    per_worker = n // NUM_WORKERS
    n_chunks = per_worker // CHUNK

    mesh = plsc.VectorSubcoreMesh(core_axis_name="c", subcore_axis_name="s")

    @functools.partial(
        pl.kernel,
        mesh=mesh,
        out_type=jax.ShapeDtypeStruct((n, PADDED_DIM), weight.dtype),
        scratch_types=[
            pltpu.VMEM((per_worker,), jnp.int32),
        ]
        + [pltpu.VMEM((CHUNK, PADDED_DIM), jnp.float32) for _ in range(NBUF)]
        + [pltpu.SemaphoreType.DMA for _ in range(2 * NBUF)],
    )
    def gather_k(table_hbm, idx_hbm, out_hbm, idx_v, *scratch):
        bufs = scratch[:NBUF]
        gsems = scratch[NBUF : 2 * NBUF]
        wsems = scratch[2 * NBUF :]
        wid = lax.axis_index("s") * NUM_CORES + lax.axis_index("c")
        base = wid * per_worker
        pltpu.sync_copy(idx_hbm.at[pl.ds(base, per_worker)], idx_v)

        def start_gather(c):
            b = c % NBUF
            return pltpu.async_copy(
                table_hbm.at[idx_v.at[pl.ds(c * CHUNK, CHUNK)]], bufs[b], gsems[b]
            )

        gh = [None] * NBUF
        wr = [None] * NBUF
        for c in range(NBUF - 1):
            gh[c % NBUF] = start_gather(c)
        for c in range(n_chunks):
            b = c % NBUF
            nxt = c + NBUF - 1
            if nxt < n_chunks:
                nb = nxt % NBUF
                if wr[nb] is not None:
                    wr[nb].wait()
                gh[nb] = start_gather(nxt)
            gh[b].wait()
            wr[b] = pltpu.async_copy(
                bufs[b], out_hbm.at[pl.ds(base + c * CHUNK, CHUNK)], wsems[b]
            )
        for w in wr:
            if w is not None:
                w.wait()

    out = gather_k(w128, idx)
    return out[:, :EMBEDDING_DIM].reshape(batch, seq, EMBEDDING_DIM)
